# baseline (speedup 1.0000x reference)
import functools

import jax
import jax.numpy as jnp
from jax import lax
from jax.experimental import pallas as pl
from jax.experimental.pallas import tpu as pltpu
from jax.experimental.pallas import tpu_sc as plsc

_N_SPECIAL = 8
_PAD_IDX = _N_SPECIAL

_NC = 2
_NS = 16
_NW = _NC * _NS
_L = 16


@functools.lru_cache(maxsize=None)
def _build(bsz: int, seq: int, d: int, n_vocab: int):
    dp = -(-d // _L) * _L + 1
    tab_words = -(-n_vocab * dp // _L) * _L
    assert bsz % (_NW * _L) == 0 and seq % 2 == 0
    b_per_w = bsz // _NW
    n_grp = b_per_w // _L
    mesh = plsc.VectorSubcoreMesh(core_axis_name="c", subcore_axis_name="s")

    @functools.partial(
        pl.kernel,
        mesh=mesh,
        out_type=jax.ShapeDtypeStruct((seq, d, bsz), jnp.float32),
        scratch_types=[
            pltpu.VMEM((b_per_w * seq,), jnp.int32),
            pltpu.VMEM((tab_words,), jnp.float32),
            pltpu.VMEM((2, -(-d // 8) * 8, b_per_w), jnp.float32),
            pltpu.SemaphoreType.DMA,
            pltpu.SemaphoreType.DMA,
        ],
        compiler_params=pltpu.CompilerParams(
            use_tc_tiling_on_sc=True, needs_layout_passes=False),
    )
    def emb(idx_hbm, tab_hbm, out_hbm, idx_v, tab_v, pack_v, osem0, osem1):
        wid = lax.axis_index("s") * _NC + lax.axis_index("c")
        base = wid * b_per_w
        pltpu.sync_copy(idx_hbm.at[pl.ds(base * seq, b_per_w * seq)], idx_v)
        pltpu.sync_copy(tab_hbm, tab_v)

        zeros = jnp.zeros((_L,), jnp.float32)
        iota = lax.iota(jnp.int32, _L)
        for r in (0, _PAD_IDX):
            for k in range(-(-d // _L)):
                plsc.store_scatter(
                    tab_v, [r * dp + k * _L + iota], zeros)
        osems = (osem0, osem1)

        def outer(ll, carry):
            for b in range(2):
                l = ll * 2 + b

                @pl.when(ll > 0)
                def _drain():
                    pltpu.make_async_copy(
                        pack_v.at[b].at[pl.ds(0, d)],
                        out_hbm.at[0].at[:, pl.ds(0, b_per_w)],
                        osems[b],
                    ).wait()

                srcbs = []
                for g in range(n_grp):
                    vidx = plsc.load_gather(
                        idx_v, [(g * _L + iota) * seq + l])
                    srcbs.append(vidx * dp)

                @plsc.parallel_loop(0, d, unroll=8)
                def _col(c):
                    for g in range(n_grp):
                        v = plsc.load_gather(tab_v, [srcbs[g] + c])
                        pack_v[b, c, pl.ds(g * _L, _L)] = v

                pltpu.async_copy(
                    pack_v.at[b].at[pl.ds(0, d)],
                    out_hbm.at[l].at[:, pl.ds(base, b_per_w)],
                    osems[b],
                )
            return carry

        lax.fori_loop(0, seq // 2, outer, 0)
        for b in range(2):
            pltpu.make_async_copy(
                pack_v.at[b].at[pl.ds(0, d)],
                out_hbm.at[0].at[:, pl.ds(0, b_per_w)],
                osems[b],
            ).wait()

    return emb


def kernel(inputs, embs_weight):
    bsz, seq = inputs.shape
    n_vocab, d = embs_weight.shape
    dp = -(-d // _L) * _L + 1
    tab_words = -(-n_vocab * dp // _L) * _L
    tab_flat = jnp.pad(embs_weight, ((0, 0), (0, dp - d))).reshape(-1)
    tab_flat = jnp.pad(tab_flat, (0, tab_words - n_vocab * dp))
    out = _build(bsz, seq, d, n_vocab)(inputs.reshape(-1), tab_flat)
    return jnp.transpose(out, (2, 0, 1))
